# baseline (speedup 1.0000x reference)
import functools

import jax
import jax.numpy as jnp
from jax import lax
from jax.experimental import pallas as pl
from jax.experimental.pallas import tpu as pltpu
from jax.experimental.pallas import tpu_sc as plsc

H = 8
N = 10000
E = 320000
C = 128
NC = 2
NS = 16
NW = NC * NS
L = 16
EPW = E // NW
K = 128
NSUB = 80
EPWP = NSUB * K
NSUBH = NSUB // 2
EH = NSUBH * K
NEG_INF = -3.4e38


def _phase_a_body(x_ref, w_ref, att_ref, xt_ref, p_ref, q_ref):
    xt = lax.dot_general(
        x_ref[...], w_ref[0],
        (((1,), (0,)), ((), ())),
        preferred_element_type=jnp.float32,
    )
    xt_ref[0] = xt
    att = att_ref[0, 0]
    p_ref[0, 0] = lax.dot_general(
        xt, att[:C], (((1,), (0,)), ((), ())),
        preferred_element_type=jnp.float32)
    q_ref[0, 0] = lax.dot_general(
        xt, att[C:], (((1,), (0,)), ((), ())),
        preferred_element_type=jnp.float32)


def _phase_a(x, w, att3):
    return pl.pallas_call(
        _phase_a_body,
        grid=(H,),
        in_specs=[
            pl.BlockSpec((N, C), lambda h: (0, 0)),
            pl.BlockSpec((1, C, C), lambda h: (h, 0, 0)),
            pl.BlockSpec((1, 1, 2 * C), lambda h: (h, 0, 0)),
        ],
        out_specs=[
            pl.BlockSpec((1, N, C), lambda h: (h, 0, 0)),
            pl.BlockSpec((1, 1, N), lambda h: (h, 0, 0)),
            pl.BlockSpec((1, 1, N), lambda h: (h, 0, 0)),
        ],
        out_shape=[
            jax.ShapeDtypeStruct((H, N, C), jnp.float32),
            jax.ShapeDtypeStruct((H, 1, N), jnp.float32),
            jax.ShapeDtypeStruct((H, 1, N), jnp.float32),
        ],
    )(x, w, att3)


@functools.cache
def _phase_b_kernel():
  mesh = plsc.VectorSubcoreMesh(
      core_axis_name="c", subcore_axis_name="s",
      num_cores=NC, num_subcores=NS)
  return functools.partial(
      pl.kernel,
      out_type=[
          jax.ShapeDtypeStruct((H * E,), jnp.float32),
          jax.ShapeDtypeStruct((NW * H * 2 * L,), jnp.float32),
      ],
      mesh=mesh,
      compiler_params=pltpu.CompilerParams(needs_layout_passes=False),
      scratch_types=[
          pltpu.VMEM((EPW,), jnp.int32),
          pltpu.VMEM((EPW,), jnp.int32),
          pltpu.VMEM((N,), jnp.float32),
          pltpu.VMEM((N,), jnp.float32),
          pltpu.VMEM((EPW,), jnp.float32),
          pltpu.VMEM((2 * L,), jnp.float32),
      ],
  )(_phase_b_body)


def _phase_b_body(edge_hbm, p_hbm, q_hbm, e_hbm, stats_hbm,
                  rbuf, cbuf, pbuf, qbuf, sbuf, stbuf):
    cid = lax.axis_index("c")
    sid = lax.axis_index("s")
    wid = sid * NC + cid
    base = wid * EPW
    pltpu.sync_copy(edge_hbm.at[pl.ds(base, EPW)], rbuf)
    pltpu.sync_copy(edge_hbm.at[pl.ds(E + base, EPW)], cbuf)

    def head_body(h, carry):
        pltpu.sync_copy(p_hbm.at[h, 0], pbuf)
        pltpu.sync_copy(q_hbm.at[h, 0], qbuf)

        def pass1(j, m):
            r16 = rbuf[pl.ds(j * L, L)]
            c16 = cbuf[pl.ds(j * L, L)]
            s = plsc.load_gather(pbuf, [r16]) + plsc.load_gather(qbuf, [c16])
            s = jnp.maximum(s, s * jnp.float32(0.01))
            sbuf[pl.ds(j * L, L)] = s
            return jnp.maximum(m, s)

        m = lax.fori_loop(0, EPW // L, pass1,
                          jnp.full((L,), NEG_INF, jnp.float32))

        def pass2(j, z):
            e = jnp.exp(sbuf[pl.ds(j * L, L)] - m)
            sbuf[pl.ds(j * L, L)] = e
            return z + e

        z = lax.fori_loop(0, EPW // L, pass2, jnp.zeros((L,), jnp.float32))
        pltpu.sync_copy(sbuf, e_hbm.at[pl.ds(h * E + base, EPW)])
        stbuf[pl.ds(0, L)] = m
        stbuf[pl.ds(L, L)] = z
        pltpu.sync_copy(stbuf, stats_hbm.at[pl.ds((h * NW + wid) * 2 * L,
                                                  2 * L)])
        return carry

    lax.fori_loop(0, H, head_body, 0)


@functools.cache
def _phase_c_kernel():
  mesh = plsc.VectorSubcoreMesh(
      core_axis_name="c", subcore_axis_name="s",
      num_cores=NC, num_subcores=NS)
  return functools.partial(
      pl.kernel,
      out_type=jax.ShapeDtypeStruct((NC, N, C), jnp.float32),
      mesh=mesh,
      compiler_params=pltpu.CompilerParams(needs_layout_passes=False),
      scratch_types=[
          pltpu.VMEM((NSUB, K), jnp.int32),
          pltpu.VMEM((EPWP,), jnp.int32),
          pltpu.VMEM((EPWP,), jnp.float32),
          pltpu.VMEM((K, C), jnp.float32),
          pltpu.VMEM((NW * 2 * L,), jnp.float32),
          pltpu.VMEM_SHARED((N, C), jnp.float32),
          pltpu.SemaphoreType.DMA,
      ],
  )(_phase_c_body)


def _phase_c_body(xtf_hbm, e_hbm, rows_hbm, cols_hbm, stats_hbm, zeros_hbm,
                  out_hbm, rbuf2, cbuf, wbuf, gbuf, statsb, acc, sem):
    cid = lax.axis_index("c")
    sid = lax.axis_index("s")
    wid = sid * NC + cid
    base = wid * EPW
    rpt = (N // NS) & ~7
    tail = N - rpt * NS
    tbase = sid * rpt

    pltpu.sync_copy(zeros_hbm.at[pl.ds(tbase, rpt)],
                    acc.at[pl.ds(tbase, rpt)])

    @pl.when(sid == NS - 1)
    def _zero_tail():
        pltpu.sync_copy(zeros_hbm.at[pl.ds(rpt * NS, tail)],
                        acc.at[pl.ds(rpt * NS, tail)])

    pltpu.sync_copy(rows_hbm.at[wid], rbuf2)
    pltpu.sync_copy(cols_hbm.at[wid, 0], cbuf)

    for t in range(EPW // L, EPWP // L):
        wbuf[pl.ds(t * L, L)] = jnp.zeros((L,), jnp.float32)

    plsc.subcore_barrier()

    def head_body(h, carry):
        pltpu.sync_copy(stats_hbm.at[pl.ds(h * NW * 2 * L, NW * 2 * L)],
                        statsb)

        def _st(w, i):
            return statsb[pl.ds((w * 2 + i) * L, L)]

        def mred(w, m):
            return jnp.maximum(m, _st(w, 0))

        mv = lax.fori_loop(0, NW, mred, jnp.full((L,), NEG_INF, jnp.float32))
        msplat = jnp.full((L,), jnp.max(mv), jnp.float32)

        def zred(w, z):
            return z + _st(w, 1) * jnp.exp(_st(w, 0) - msplat)

        zv = lax.fori_loop(0, NW, zred, jnp.zeros((L,), jnp.float32))
        zsum = jnp.sum(zv)
        scale = jnp.exp(_st(wid, 0) - msplat) / (
            jnp.full((L,), zsum * jnp.float32(H * H), jnp.float32))

        pltpu.sync_copy(e_hbm.at[pl.ds(h * E + base, EPW)],
                        wbuf.at[pl.ds(0, EPW)])

        def wscale(t, carry2):
            wbuf[pl.ds(t * L, L)] = wbuf[pl.ds(t * L, L)] * scale
            return carry2

        lax.fori_loop(0, EPWP // L, wscale, 0)

        @pl.when(h > 0)
        def _bump_cols():
            def cbump(t, carry2):
                cbuf[pl.ds(t * L, L)] = (
                    cbuf[pl.ds(t * L, L)] + jnp.full((L,), N, jnp.int32))
                return carry2

            lax.fori_loop(0, EPWP // L, cbump, 0)

        ones = jnp.full((L,), 1, jnp.int32)

        def sub_body(j, carry2):
            pltpu.async_copy(
                xtf_hbm.at[cbuf.at[pl.ds(j * K, K)]], gbuf, sem).wait()

            def scale_row(i, idxv):
                w = plsc.load_gather(wbuf, [idxv])
                for g in range(C // L):
                    gbuf.at[i][pl.ds(g * L, L)] = (
                        gbuf.at[i][pl.ds(g * L, L)] * w)
                return idxv + ones

            lax.fori_loop(0, K, scale_row,
                          jnp.full((L,), j * K, jnp.int32))
            pltpu.sync_copy(gbuf, acc.at[rbuf2.at[j]], add=True)
            return carry2

        lax.fori_loop(0, NSUB, sub_body, 0)
        return carry

    lax.fori_loop(0, H, head_body, 0)
    plsc.subcore_barrier()

    pltpu.sync_copy(acc.at[pl.ds(tbase, rpt)],
                    out_hbm.at[cid, pl.ds(tbase, rpt)])

    @pl.when(sid == NS - 1)
    def _dump_tail():
        pltpu.sync_copy(acc.at[pl.ds(rpt * NS, tail)],
                        out_hbm.at[cid, pl.ds(rpt * NS, tail)])


def _combine_body(p_ref, o_ref):
    o_ref[...] = p_ref[0] + p_ref[1]


def _combine(partial):
    blk = N // 5
    return pl.pallas_call(
        _combine_body,
        grid=(5,),
        in_specs=[pl.BlockSpec((NC, blk, C), lambda i: (0, i, 0))],
        out_specs=pl.BlockSpec((blk, C), lambda i: (i, 0)),
        out_shape=jax.ShapeDtypeStruct((N, C), jnp.float32),
    )(partial)


def kernel(x, edge_index, meta_weight, meta_att):
    att3 = meta_att.reshape(H, 1, 2 * C)
    eflat = edge_index.reshape(2 * E)
    epad = jnp.pad(edge_index.reshape(2, NW, EPW),
                   ((0, 0), (0, 0), (0, EPWP - EPW)))
    rows_pad = epad[0].reshape(NW, NSUB, K)
    cols_pad = epad[1].reshape(NW, 1, EPWP)
    xt, p, q = _phase_a(x, meta_weight, att3)
    ebuf, stats = _phase_b_kernel()(eflat, p, q)
    xtf = xt.reshape(H * N, C)
    zeros = jnp.zeros((N, C), jnp.float32)
    partial = _phase_c_kernel()(xtf, ebuf, rows_pad, cols_pad, stats, zeros)
    return _combine(partial)


# speedup vs baseline: 1.9932x; 1.9932x over previous
import functools

import jax
import jax.numpy as jnp
from jax import lax
from jax.experimental import pallas as pl
from jax.experimental.pallas import tpu as pltpu
from jax.experimental.pallas import tpu_sc as plsc

H = 8
N = 10000
E = 320000
C = 128
NC = 2
NS = 16
NW = NC * NS
L = 16
EPW = E // NW
K = 128
NSUB = 80
EPWP = NSUB * K
NSUBH = NSUB // 2
EH = NSUBH * K
NEG_INF = -3.4e38

_GATHER_DNUMS = lax.GatherDimensionNumbers(
    offset_dims=(), collapsed_slice_dims=(0,), start_index_map=(0,))


def _phase_a_body(x_ref, w_ref, att_ref, xt_ref, p_ref, q_ref):
    xt = lax.dot_general(
        x_ref[...], w_ref[0],
        (((1,), (0,)), ((), ())),
        preferred_element_type=jnp.float32,
    )
    xt_ref[0] = xt
    att = att_ref[0, 0]
    p_ref[0, 0] = lax.dot_general(
        xt, att[:C], (((1,), (0,)), ((), ())),
        preferred_element_type=jnp.float32)
    q_ref[0, 0] = lax.dot_general(
        xt, att[C:], (((1,), (0,)), ((), ())),
        preferred_element_type=jnp.float32)


def _phase_a(x, w, att3):
    return pl.pallas_call(
        _phase_a_body,
        grid=(H,),
        in_specs=[
            pl.BlockSpec((N, C), lambda h: (0, 0)),
            pl.BlockSpec((1, C, C), lambda h: (h, 0, 0)),
            pl.BlockSpec((1, 1, 2 * C), lambda h: (h, 0, 0)),
        ],
        out_specs=[
            pl.BlockSpec((1, N, C), lambda h: (h, 0, 0)),
            pl.BlockSpec((1, 1, N), lambda h: (h, 0, 0)),
            pl.BlockSpec((1, 1, N), lambda h: (h, 0, 0)),
        ],
        out_shape=[
            jax.ShapeDtypeStruct((H, N, C), jnp.float32),
            jax.ShapeDtypeStruct((H, 1, N), jnp.float32),
            jax.ShapeDtypeStruct((H, 1, N), jnp.float32),
        ],
    )(x, w, att3)


@functools.cache
def _phase_b_kernel():
  mesh = plsc.VectorSubcoreMesh(
      core_axis_name="c", subcore_axis_name="s",
      num_cores=NC, num_subcores=NS)
  return functools.partial(
      pl.kernel,
      out_type=[
          jax.ShapeDtypeStruct((H * E,), jnp.float32),
          jax.ShapeDtypeStruct((NW * H * 2 * L,), jnp.float32),
      ],
      mesh=mesh,
      compiler_params=pltpu.CompilerParams(needs_layout_passes=False),
      scratch_types=[
          pltpu.VMEM((EPW,), jnp.int32),
          pltpu.VMEM((EPW,), jnp.int32),
          pltpu.VMEM((N,), jnp.float32),
          pltpu.VMEM((N,), jnp.float32),
          pltpu.VMEM((EPW,), jnp.float32),
          pltpu.VMEM((2 * L,), jnp.float32),
      ],
  )(_phase_b_body)


def _phase_b_body(edge_hbm, p_hbm, q_hbm, e_hbm, stats_hbm,
                  rbuf, cbuf, pbuf, qbuf, sbuf, stbuf):
    cid = lax.axis_index("c")
    sid = lax.axis_index("s")
    wid = sid * NC + cid
    base = wid * EPW
    pltpu.sync_copy(edge_hbm.at[pl.ds(base, EPW)], rbuf)
    pltpu.sync_copy(edge_hbm.at[pl.ds(E + base, EPW)], cbuf)

    def head_body(h, carry):
        pltpu.sync_copy(p_hbm.at[h, 0], pbuf)
        pltpu.sync_copy(q_hbm.at[h, 0], qbuf)

        def pass1(j, m):
            r16 = rbuf[pl.ds(j * L, L)]
            c16 = cbuf[pl.ds(j * L, L)]
            s = plsc.load_gather(pbuf, [r16]) + plsc.load_gather(qbuf, [c16])
            s = jnp.maximum(s, s * jnp.float32(0.01))
            sbuf[pl.ds(j * L, L)] = s
            return jnp.maximum(m, s)

        m = lax.fori_loop(0, EPW // L, pass1,
                          jnp.full((L,), NEG_INF, jnp.float32))

        def pass2(j, z):
            e = jnp.exp(sbuf[pl.ds(j * L, L)] - m)
            sbuf[pl.ds(j * L, L)] = e
            return z + e

        z = lax.fori_loop(0, EPW // L, pass2, jnp.zeros((L,), jnp.float32))
        pltpu.sync_copy(sbuf, e_hbm.at[pl.ds(h * E + base, EPW)])
        stbuf[pl.ds(0, L)] = m
        stbuf[pl.ds(L, L)] = z
        pltpu.sync_copy(stbuf, stats_hbm.at[pl.ds((h * NW + wid) * 2 * L,
                                                  2 * L)])
        return carry

    lax.fori_loop(0, H, head_body, 0)


@functools.cache
def _phase_c_kernel():
  mesh = plsc.VectorSubcoreMesh(
      core_axis_name="c", subcore_axis_name="s",
      num_cores=NC, num_subcores=NS)
  return functools.partial(
      pl.kernel,
      out_type=jax.ShapeDtypeStruct((NC, N, C), jnp.float32),
      mesh=mesh,
      compiler_params=pltpu.CompilerParams(needs_layout_passes=False),
      scratch_types=[
          pltpu.VMEM((NSUB, K), jnp.int32),
          pltpu.VMEM((EPWP,), jnp.int32),
          pltpu.VMEM((EPWP,), jnp.float32),
          pltpu.VMEM((K, C), jnp.float32),
          pltpu.VMEM((NW * 2 * L,), jnp.float32),
          pltpu.VMEM_SHARED((N, C), jnp.float32),
          pltpu.SemaphoreType.DMA,
      ],
  )(_phase_c_body)


def _phase_c_body(xtf_hbm, e_hbm, rows_hbm, cols_hbm, stats_hbm, zeros_hbm,
                  out_hbm, rbuf2, cbuf, wbuf, gbuf, statsb, acc, sem):
    cid = lax.axis_index("c")
    sid = lax.axis_index("s")
    wid = sid * NC + cid
    base = wid * EPW
    rpt = (N // NS) & ~7
    tail = N - rpt * NS
    tbase = sid * rpt

    pltpu.sync_copy(zeros_hbm.at[pl.ds(tbase, rpt)],
                    acc.at[pl.ds(tbase, rpt)])

    @pl.when(sid == NS - 1)
    def _zero_tail():
        pltpu.sync_copy(zeros_hbm.at[pl.ds(rpt * NS, tail)],
                        acc.at[pl.ds(rpt * NS, tail)])

    pltpu.sync_copy(rows_hbm.at[wid], rbuf2)
    pltpu.sync_copy(cols_hbm.at[wid, 0], cbuf)

    for t in range(EPW // L, EPWP // L):
        wbuf[pl.ds(t * L, L)] = jnp.zeros((L,), jnp.float32)

    plsc.subcore_barrier()

    def head_body(h, carry):
        pltpu.sync_copy(stats_hbm.at[pl.ds(h * NW * 2 * L, NW * 2 * L)],
                        statsb)

        def _st(w, i):
            return statsb[pl.ds((w * 2 + i) * L, L)]

        def mred(w, m):
            return jnp.maximum(m, _st(w, 0))

        mv = lax.fori_loop(0, NW, mred, jnp.full((L,), NEG_INF, jnp.float32))
        msplat = jnp.full((L,), jnp.max(mv), jnp.float32)

        def zred(w, z):
            return z + _st(w, 1) * jnp.exp(_st(w, 0) - msplat)

        zv = lax.fori_loop(0, NW, zred, jnp.zeros((L,), jnp.float32))
        zsum = jnp.sum(zv)
        scale = jnp.exp(_st(wid, 0) - msplat) / (
            jnp.full((L,), zsum * jnp.float32(H * H), jnp.float32))

        pltpu.sync_copy(e_hbm.at[pl.ds(h * E + base, EPW)],
                        wbuf.at[pl.ds(0, EPW)])

        def wscale(t, carry2):
            wbuf[pl.ds(t * L, L)] = wbuf[pl.ds(t * L, L)] * scale
            return carry2

        lax.fori_loop(0, EPWP // L, wscale, 0)

        @pl.when(h > 0)
        def _bump_cols():
            def cbump(t, carry2):
                cbuf[pl.ds(t * L, L)] = (
                    cbuf[pl.ds(t * L, L)] + jnp.full((L,), N, jnp.int32))
                return carry2

            lax.fori_loop(0, EPWP // L, cbump, 0)

        def sub_body(j, carry2):
            pltpu.async_copy(
                xtf_hbm.at[cbuf.at[pl.ds(j * K, K)]], gbuf, sem).wait()

            def scale_block(b, carry3):
                w16 = wbuf[pl.ds(j * K + b * L, L)]
                for r in range(L):
                    w = lax.gather(
                        w16, jnp.full((L, 1), r, jnp.int32),
                        _GATHER_DNUMS, slice_sizes=(1,),
                        mode=lax.GatherScatterMode.PROMISE_IN_BOUNDS)
                    i = b * L + r
                    for g in range(C // L):
                        gbuf.at[i][pl.ds(g * L, L)] = (
                            gbuf.at[i][pl.ds(g * L, L)] * w)
                return carry3

            lax.fori_loop(0, K // L, scale_block, 0)
            pltpu.sync_copy(gbuf, acc.at[rbuf2.at[j]], add=True)
            return carry2

        lax.fori_loop(0, NSUB, sub_body, 0)
        return carry

    lax.fori_loop(0, H, head_body, 0)
    plsc.subcore_barrier()

    pltpu.sync_copy(acc.at[pl.ds(tbase, rpt)],
                    out_hbm.at[cid, pl.ds(tbase, rpt)])

    @pl.when(sid == NS - 1)
    def _dump_tail():
        pltpu.sync_copy(acc.at[pl.ds(rpt * NS, tail)],
                        out_hbm.at[cid, pl.ds(rpt * NS, tail)])


def _combine_body(p_ref, o_ref):
    o_ref[...] = p_ref[0] + p_ref[1]


def _combine(partial):
    blk = N // 5
    return pl.pallas_call(
        _combine_body,
        grid=(5,),
        in_specs=[pl.BlockSpec((NC, blk, C), lambda i: (0, i, 0))],
        out_specs=pl.BlockSpec((blk, C), lambda i: (i, 0)),
        out_shape=jax.ShapeDtypeStruct((N, C), jnp.float32),
    )(partial)


def kernel(x, edge_index, meta_weight, meta_att):
    att3 = meta_att.reshape(H, 1, 2 * C)
    eflat = edge_index.reshape(2 * E)
    npad = EPWP - EPW
    padv = (jnp.arange(npad, dtype=jnp.int32)[None, :] * 67
            + jnp.arange(NW, dtype=jnp.int32)[:, None] * 577) % N
    ei2 = edge_index.reshape(2, NW, EPW)
    rows_pad = jnp.concatenate([ei2[0], padv], axis=1).reshape(NW, NSUB, K)
    cols_pad = jnp.concatenate([ei2[1], padv], axis=1).reshape(NW, 1, EPWP)
    xt, p, q = _phase_a(x, meta_weight, att3)
    ebuf, stats = _phase_b_kernel()(eflat, p, q)
    xtf = xt.reshape(H * N, C)
    zeros = jnp.zeros((N, C), jnp.float32)
    partial = _phase_c_kernel()(xtf, ebuf, rows_pad, cols_pad, stats, zeros)
    return _combine(partial)


# speedup vs baseline: 2.8479x; 1.4288x over previous
import functools

import jax
import jax.numpy as jnp
from jax import lax
from jax.experimental import pallas as pl
from jax.experimental.pallas import tpu as pltpu
from jax.experimental.pallas import tpu_sc as plsc

H = 8
N = 10000
E = 320000
C = 128
NC = 2
NS = 16
NW = NC * NS
L = 16
EPW = E // NW
K = 128
NSUB = 80
EPWP = NSUB * K
NSUBH = NSUB // 2
EH = NSUBH * K
NEG_INF = -3.4e38

_GATHER_DNUMS = lax.GatherDimensionNumbers(
    offset_dims=(), collapsed_slice_dims=(0,), start_index_map=(0,))


def _phase_a_body(x_ref, w_ref, att_ref, xt_ref, p_ref, q_ref):
    xt = lax.dot_general(
        x_ref[...], w_ref[0],
        (((1,), (0,)), ((), ())),
        preferred_element_type=jnp.float32,
    )
    xt_ref[0] = xt
    att = att_ref[0, 0]
    p_ref[0, 0] = lax.dot_general(
        xt, att[:C], (((1,), (0,)), ((), ())),
        preferred_element_type=jnp.float32)
    q_ref[0, 0] = lax.dot_general(
        xt, att[C:], (((1,), (0,)), ((), ())),
        preferred_element_type=jnp.float32)


def _phase_a(x, w, att3):
    return pl.pallas_call(
        _phase_a_body,
        grid=(H,),
        in_specs=[
            pl.BlockSpec((N, C), lambda h: (0, 0)),
            pl.BlockSpec((1, C, C), lambda h: (h, 0, 0)),
            pl.BlockSpec((1, 1, 2 * C), lambda h: (h, 0, 0)),
        ],
        out_specs=[
            pl.BlockSpec((1, N, C), lambda h: (h, 0, 0)),
            pl.BlockSpec((1, 1, N), lambda h: (h, 0, 0)),
            pl.BlockSpec((1, 1, N), lambda h: (h, 0, 0)),
        ],
        out_shape=[
            jax.ShapeDtypeStruct((H, N, C), jnp.float32),
            jax.ShapeDtypeStruct((H, 1, N), jnp.float32),
            jax.ShapeDtypeStruct((H, 1, N), jnp.float32),
        ],
    )(x, w, att3)


@functools.cache
def _phase_b_kernel():
  mesh = plsc.VectorSubcoreMesh(
      core_axis_name="c", subcore_axis_name="s",
      num_cores=NC, num_subcores=NS)
  return functools.partial(
      pl.kernel,
      out_type=[
          jax.ShapeDtypeStruct((H * E,), jnp.float32),
          jax.ShapeDtypeStruct((NW * H * 2 * L,), jnp.float32),
      ],
      mesh=mesh,
      compiler_params=pltpu.CompilerParams(needs_layout_passes=False),
      scratch_types=[
          pltpu.VMEM((EPW,), jnp.int32),
          pltpu.VMEM((EPW,), jnp.int32),
          pltpu.VMEM((N,), jnp.float32),
          pltpu.VMEM((N,), jnp.float32),
          pltpu.VMEM((EPW,), jnp.float32),
          pltpu.VMEM((2 * L,), jnp.float32),
      ],
  )(_phase_b_body)


def _phase_b_body(edge_hbm, p_hbm, q_hbm, e_hbm, stats_hbm,
                  rbuf, cbuf, pbuf, qbuf, sbuf, stbuf):
    cid = lax.axis_index("c")
    sid = lax.axis_index("s")
    wid = sid * NC + cid
    base = wid * EPW
    pltpu.sync_copy(edge_hbm.at[pl.ds(base, EPW)], rbuf)
    pltpu.sync_copy(edge_hbm.at[pl.ds(E + base, EPW)], cbuf)

    def head_body(h, carry):
        pltpu.sync_copy(p_hbm.at[h, 0], pbuf)
        pltpu.sync_copy(q_hbm.at[h, 0], qbuf)

        def pass1(j, m):
            r16 = rbuf[pl.ds(j * L, L)]
            c16 = cbuf[pl.ds(j * L, L)]
            s = plsc.load_gather(pbuf, [r16]) + plsc.load_gather(qbuf, [c16])
            s = jnp.maximum(s, s * jnp.float32(0.01))
            sbuf[pl.ds(j * L, L)] = s
            return jnp.maximum(m, s)

        m = lax.fori_loop(0, EPW // L, pass1,
                          jnp.full((L,), NEG_INF, jnp.float32))

        def pass2(j, z):
            e = jnp.exp(sbuf[pl.ds(j * L, L)] - m)
            sbuf[pl.ds(j * L, L)] = e
            return z + e

        z = lax.fori_loop(0, EPW // L, pass2, jnp.zeros((L,), jnp.float32))
        pltpu.sync_copy(sbuf, e_hbm.at[pl.ds(h * E + base, EPW)])
        stbuf[pl.ds(0, L)] = m
        stbuf[pl.ds(L, L)] = z
        pltpu.sync_copy(stbuf, stats_hbm.at[pl.ds((h * NW + wid) * 2 * L,
                                                  2 * L)])
        return carry

    lax.fori_loop(0, H, head_body, 0)


@functools.cache
def _phase_c_kernel():
  mesh = plsc.VectorSubcoreMesh(
      core_axis_name="c", subcore_axis_name="s",
      num_cores=NC, num_subcores=NS)
  return functools.partial(
      pl.kernel,
      out_type=jax.ShapeDtypeStruct((NC, N, C), jnp.float32),
      mesh=mesh,
      compiler_params=pltpu.CompilerParams(needs_layout_passes=False),
      scratch_types=[
          pltpu.VMEM((NSUBH, K), jnp.int32),
          pltpu.VMEM((EH,), jnp.int32),
          pltpu.VMEM((EH,), jnp.float32),
          pltpu.VMEM((K, C), jnp.float32),
          pltpu.VMEM((K, C), jnp.float32),
          pltpu.VMEM((NW * 2 * L,), jnp.float32),
          pltpu.VMEM_SHARED((N, C), jnp.float32),
          pltpu.SemaphoreType.DMA,
          pltpu.SemaphoreType.DMA,
          pltpu.SemaphoreType.DMA,
          pltpu.SemaphoreType.DMA,
      ],
  )(_phase_c_body)


def _phase_c_body(xtf_hbm, e_hbm, rows_hbm, cols_hbm, stats_hbm, zeros_hbm,
                  out_hbm, rbuf2, cbuf, wbuf, gbufa, gbufb, statsb, acc,
                  gsema, gsemb, ssema, ssemb):
    cid = lax.axis_index("c")
    sid = lax.axis_index("s")
    wid = sid * NC + cid
    base = wid * EPW
    rpt = (N // NS) & ~7
    tail = N - rpt * NS
    tbase = sid * rpt

    pltpu.sync_copy(zeros_hbm.at[pl.ds(tbase, rpt)],
                    acc.at[pl.ds(tbase, rpt)])

    @pl.when(sid == NS - 1)
    def _zero_tail():
        pltpu.sync_copy(zeros_hbm.at[pl.ds(rpt * NS, tail)],
                        acc.at[pl.ds(rpt * NS, tail)])

    plsc.subcore_barrier()

    def head_body(h, carry):
        pltpu.sync_copy(stats_hbm.at[pl.ds(h * NW * 2 * L, NW * 2 * L)],
                        statsb)

        def _st(w, i):
            return statsb[pl.ds((w * 2 + i) * L, L)]

        def mred(w, m):
            return jnp.maximum(m, _st(w, 0))

        mv = lax.fori_loop(0, NW, mred, jnp.full((L,), NEG_INF, jnp.float32))
        msplat = jnp.full((L,), jnp.max(mv), jnp.float32)

        def zred(w, z):
            return z + _st(w, 1) * jnp.exp(_st(w, 0) - msplat)

        zv = lax.fori_loop(0, NW, zred, jnp.zeros((L,), jnp.float32))
        zsum = jnp.sum(zv)
        scale = jnp.exp(_st(wid, 0) - msplat) / (
            jnp.full((L,), zsum * jnp.float32(H * H), jnp.float32))

        hsplat = jnp.full((L,), h * N, jnp.int32)

        def start_gather(j, gbuf, gsem):
            pltpu.async_copy(
                xtf_hbm.at[cbuf.at[pl.ds(j * K, K)]], gbuf, gsem)

        def wait_gather(gbuf, gsem):
            pltpu.make_async_copy(
                xtf_hbm.at[cbuf.at[pl.ds(0, K)]], gbuf, gsem).wait()

        def scale_buf(j, gbuf):
            def scale_block(b, carry3):
                w16 = wbuf[pl.ds(j * K + b * L, L)]
                for r in range(L):
                    w = lax.gather(
                        w16, jnp.full((L, 1), r, jnp.int32),
                        _GATHER_DNUMS, slice_sizes=(1,),
                        mode=lax.GatherScatterMode.PROMISE_IN_BOUNDS)
                    i = b * L + r
                    for g in range(C // L):
                        gbuf.at[i][pl.ds(g * L, L)] = (
                            gbuf.at[i][pl.ds(g * L, L)] * w)
                return carry3

            lax.fori_loop(0, K // L, scale_block, 0)

        def start_scatter(j, gbuf, ssem):
            pltpu.async_copy(gbuf, acc.at[rbuf2.at[j]], ssem, add=True)

        def wait_scatter(gbuf, ssem):
            pltpu.make_async_copy(gbuf, acc.at[rbuf2.at[0]], ssem).wait()

        for half in range(2):
            ebase = half * EH
            nreal = (EPW - EH) if half else EH

            pltpu.sync_copy(rows_hbm.at[wid, half], rbuf2)
            pltpu.sync_copy(cols_hbm.at[wid, 0, pl.ds(ebase, EH)], cbuf)
            pltpu.sync_copy(e_hbm.at[pl.ds(h * E + base + ebase, nreal)],
                            wbuf.at[pl.ds(0, nreal)])
            for t in range(nreal // L, EH // L):
                wbuf[pl.ds(t * L, L)] = jnp.zeros((L,), jnp.float32)

            def stage(t, carry2):
                wbuf[pl.ds(t * L, L)] = wbuf[pl.ds(t * L, L)] * scale
                cbuf[pl.ds(t * L, L)] = cbuf[pl.ds(t * L, L)] + hsplat
                return carry2

            lax.fori_loop(0, EH // L, stage, 0)

            start_gather(0, gbufa, gsema)
            start_gather(1, gbufb, gsemb)

            def sub_body(t, carry2):
                ja = 2 * t
                jb = 2 * t + 1

                @pl.when(t > 0)
                def _prep_b():
                    wait_scatter(gbufb, ssemb)
                    start_gather(jb, gbufb, gsemb)

                wait_gather(gbufa, gsema)
                scale_buf(ja, gbufa)
                start_scatter(ja, gbufa, ssema)
                wait_gather(gbufb, gsemb)
                scale_buf(jb, gbufb)
                start_scatter(jb, gbufb, ssemb)
                wait_scatter(gbufa, ssema)

                @pl.when(ja + 2 < NSUBH)
                def _next_a():
                    start_gather(ja + 2, gbufa, gsema)

                return carry2

            lax.fori_loop(0, NSUBH // 2, sub_body, 0)
            wait_scatter(gbufb, ssemb)
        return carry

    lax.fori_loop(0, H, head_body, 0)
    plsc.subcore_barrier()

    pltpu.sync_copy(acc.at[pl.ds(tbase, rpt)],
                    out_hbm.at[cid, pl.ds(tbase, rpt)])

    @pl.when(sid == NS - 1)
    def _dump_tail():
        pltpu.sync_copy(acc.at[pl.ds(rpt * NS, tail)],
                        out_hbm.at[cid, pl.ds(rpt * NS, tail)])


def _combine_body(p_ref, o_ref):
    o_ref[...] = p_ref[0] + p_ref[1]


def _combine(partial):
    blk = N // 5
    return pl.pallas_call(
        _combine_body,
        grid=(5,),
        in_specs=[pl.BlockSpec((NC, blk, C), lambda i: (0, i, 0))],
        out_specs=pl.BlockSpec((blk, C), lambda i: (i, 0)),
        out_shape=jax.ShapeDtypeStruct((N, C), jnp.float32),
    )(partial)


def kernel(x, edge_index, meta_weight, meta_att):
    att3 = meta_att.reshape(H, 1, 2 * C)
    eflat = edge_index.reshape(2 * E)
    npad = EPWP - EPW
    padv = (jnp.arange(npad, dtype=jnp.int32)[None, :] * 67
            + jnp.arange(NW, dtype=jnp.int32)[:, None] * 577) % N
    ei2 = edge_index.reshape(2, NW, EPW)
    rows_pad = jnp.concatenate([ei2[0], padv], axis=1).reshape(
        NW, 2, NSUBH, K)
    cols_pad = jnp.concatenate([ei2[1], padv], axis=1).reshape(NW, 1, EPWP)
    xt, p, q = _phase_a(x, meta_weight, att3)
    ebuf, stats = _phase_b_kernel()(eflat, p, q)
    xtf = xt.reshape(H * N, C)
    zeros = jnp.zeros((N, C), jnp.float32)
    partial = _phase_c_kernel()(xtf, ebuf, rows_pad, cols_pad, stats, zeros)
    return _combine(partial)
